# baseline (speedup 1.0000x reference)
import jax
import jax.numpy as jnp
from jax.experimental import pallas as pl
from jax.experimental.pallas import tpu as pltpu

_EPS = 1e-5
_B_BLK = 32
_LANE = 128


def _pad_to(n, m):
    return (n + m - 1) // m * m


def _banded_cat(w_dyfirst, w_in, w_out, k, kpad):
    c_out, c_in = w_dyfirst.shape[0], w_dyfirst.shape[1]
    d = jnp.arange(w_in)[:, None] - jnp.arange(w_out)[None, :]
    wt = w_dyfirst.transpose(2, 1, 0, 3)
    band = jnp.zeros((k, c_in, w_in, c_out, w_out), w_dyfirst.dtype)
    for dx in range(k):
        onehot = (d == dx).astype(w_dyfirst.dtype)
        band = band + (wt[:, :, None, :, dx, None]
                       * onehot[None, None, :, None, :])
    band = band.reshape(k, c_in * w_in, c_out * w_out)
    band = jnp.pad(band, ((0, 0), (0, kpad - c_in * w_in), (0, 0)))
    return band.reshape(k * kpad, c_out * w_out)


def _col_pool_sel(c, w_out, npad, dtype):
    wp = w_out // 2
    rr = jnp.arange(c * w_out)[:, None]
    cc = jnp.arange(npad)[None, :]
    valid = cc < c * wp
    base = jnp.where(valid, (cc // wp) * w_out + 2 * (cc % wp), -1)
    s0 = (rr == base).astype(dtype)
    s1 = (rr == base + 1).astype(dtype) * valid.astype(dtype)
    return jnp.concatenate([s0, s1], axis=1)


def _transpose_kernel(x, bf16):
    b, c_in, h_in, w_in = x.shape

    def tbody(x_ref, o_ref):
        o_ref[...] = x_ref[...].astype(bf16)

    out = pl.pallas_call(
        tbody,
        out_shape=jax.ShapeDtypeStruct((h_in, b, c_in, w_in), bf16),
        grid=(h_in,),
        in_specs=[pl.BlockSpec((b, c_in, None, w_in), lambda h: (0, 0, h, 0))],
        out_specs=pl.BlockSpec((None, b, c_in, w_in), lambda h: (h, 0, 0, 0)),
        compiler_params=pltpu.CompilerParams(
            dimension_semantics=("arbitrary",)),
    )(x)
    return out.reshape(h_in, b, c_in * w_in)


def _make_body(k, c_in, c1, c2, h_in, w_in, n_hid, act_dim):
    h1o, w1o = h_in - k + 1, w_in - k + 1
    h1p, w1p = h1o // 2, w1o // 2
    h2o, w2o = h1p - k + 1, w1p - k + 1
    h2p, w2p = h2o // 2, w2o // 2
    bb = _B_BLK
    kw1 = c_in * w_in
    k1pad = _pad_to(kw1, _LANE)
    n1pad = _pad_to(c1 * w1p, _LANE)
    kw2 = n1pad
    bf16 = jnp.bfloat16
    f32 = jnp.float32
    my1 = 16
    my2 = 12

    def body(x_ref, w1b_ref, sh1_ref, s1_ref,
             w2b_ref, sh2_ref, c20_ref, c21_ref,
             wp_ref, bfc_ref, wf1_ref, bf1_ref, wf2_ref, bf2_ref, o_ref):
        yc_chunks = []
        for m in range(h1o // my1):
            base = m * my1
            slabs = [jnp.pad(
                x_ref[base + dy:base + dy + my1].reshape(my1 * bb, kw1),
                ((0, 0), (0, k1pad - kw1))) for dy in range(k)]
            im1 = jnp.concatenate(slabs, axis=1)
            acc = jnp.dot(im1, w1b_ref[...], preferred_element_type=f32)
            y = jnp.maximum(acc + sh1_ref[...], 0.0).astype(bf16)
            yca = jnp.dot(y, s1_ref[...], preferred_element_type=f32)
            yc_chunks.append(
                jnp.maximum(yca[:, :n1pad], yca[:, n1pad:]).astype(bf16))
        yc = jnp.concatenate(yc_chunks, axis=0)
        yr = yc.reshape(h1p, 2, bb, n1pad)
        p1 = jnp.maximum(yr[:, 0], yr[:, 1])

        yc2_chunks = []
        for m in range(h2o // my2):
            base = m * my2
            slabs2 = [p1[base + dy:base + dy + my2].reshape(my2 * bb, kw2)
                      for dy in range(k)]
            im2 = jnp.concatenate(slabs2, axis=1)
            acc2 = jnp.dot(im2, w2b_ref[...], preferred_element_type=f32)
            y2 = jnp.maximum(acc2 + sh2_ref[...], 0.0).astype(bf16)
            yc2_chunks.append(jnp.maximum(
                jnp.dot(y2, c20_ref[...], preferred_element_type=f32),
                jnp.dot(y2, c21_ref[...], preferred_element_type=f32)
            ).astype(bf16))
        yc2 = jnp.concatenate(yc2_chunks, axis=0)
        yr2 = yc2.reshape(h2p, 2, bb, c2 * w2p)
        p2 = jnp.maximum(yr2[:, 0], yr2[:, 1])

        hid = jnp.broadcast_to(bfc_ref[...], (bb, n_hid))
        for h in range(h2p):
            hid = hid + jnp.dot(p2[h], wp_ref[h], preferred_element_type=f32)
        hid = jnp.maximum(hid, 0.0).astype(bf16)
        hid = jnp.maximum(
            jnp.dot(hid, wf1_ref[...], preferred_element_type=f32)
            + bf1_ref[...], 0.0).astype(bf16)
        o_ref[...] = (jnp.dot(hid, wf2_ref[...], preferred_element_type=f32)
                      + bf2_ref[...])

    return body, (h1o, w1o, h1p, w1p, h2o, w2o, h2p, w2p, k1pad, n1pad)


def kernel(x, w1, b1, bn1_gamma, bn1_beta, bn1_mean, bn1_var,
           w2, b2, bn2_gamma, bn2_beta, bn2_mean, bn2_var,
           fc_w, fc_b, fc1_w, fc1_b, fc2_w, fc2_b):
    b, c_in, h_in, w_in = x.shape
    k = w1.shape[-1]
    c1 = w1.shape[0]
    c2 = w2.shape[0]
    act_dim = fc2_w.shape[0]
    n_hid = fc_w.shape[0]

    body, (h1o, w1o, h1p, w1p, h2o, w2o, h2p, w2p, k1pad, n1pad) = _make_body(
        k, c_in, c1, c2, h_in, w_in, n_hid, act_dim)

    f32 = jnp.float32
    bf16 = jnp.bfloat16
    s1 = bn1_gamma / jnp.sqrt(bn1_var + _EPS)
    w1f = w1 * s1[:, None, None, None]
    sh1 = bn1_beta + (b1 - bn1_mean) * s1
    s2 = bn2_gamma / jnp.sqrt(bn2_var + _EPS)
    w2f = w2 * s2[:, None, None, None]
    sh2 = bn2_beta + (b2 - bn2_mean) * s2

    w1cat = _banded_cat(w1f, w_in, w1o, k, k1pad).astype(bf16)
    w2cat = _banded_cat(w2f, w1p, w2o, k, n1pad).astype(bf16)
    sh1rep = jnp.broadcast_to(sh1[:, None], (c1, w1o)).reshape(1, c1 * w1o)
    sh2rep = jnp.broadcast_to(sh2[:, None], (c2, w2o)).reshape(1, c2 * w2o)
    s1sel = _col_pool_sel(c1, w1o, n1pad, bf16)
    c2half = c2 * (w2o // 2)
    rr = jnp.arange(c2 * w2o)[:, None]
    cc = jnp.arange(c2half)[None, :]
    base2 = (cc // (w2o // 2)) * w2o + 2 * (cc % (w2o // 2))
    c20 = (rr == base2).astype(bf16)
    c21 = (rr == base2 + 1).astype(bf16)

    wp = (fc_w.T.reshape(c2, h2p, w2p, n_hid)
          .transpose(1, 0, 2, 3).reshape(h2p, c2 * w2p, n_hid)).astype(bf16)
    wf1 = fc1_w.T.astype(bf16)
    wf2 = fc2_w.T.astype(bf16)
    bfc = fc_b[None, :]
    bf1 = fc1_b[None, :]
    bf2 = fc2_b[None, :]

    x_t = x.astype(bf16).transpose(2, 0, 1, 3).reshape(h_in, b, c_in * w_in)

    def const_spec(t):
        return pl.BlockSpec(t.shape, lambda i: (0,) * t.ndim)

    n_blk = b // _B_BLK
    flops = 2 * b * (h1o * (k * k1pad) * (c1 * w1o)
                     + h2o * (k * n1pad) * (c2 * w2o)
                     + c2 * h2p * w2p * n_hid + n_hid * 84 + 84 * act_dim)
    bytes_accessed = 2 * (x_t.size + w1cat.size + w2cat.size
                          + wp.size + wf1.size + wf2.size) + 4 * b * act_dim

    out = pl.pallas_call(
        body,
        out_shape=jax.ShapeDtypeStruct((b, act_dim), f32),
        grid=(n_blk,),
        in_specs=[
            pl.BlockSpec((h_in, _B_BLK, c_in * w_in), lambda i: (0, i, 0)),
            const_spec(w1cat), const_spec(sh1rep), const_spec(s1sel),
            const_spec(w2cat), const_spec(sh2rep),
            const_spec(c20), const_spec(c21),
            const_spec(wp), const_spec(bfc),
            const_spec(wf1), const_spec(bf1),
            const_spec(wf2), const_spec(bf2),
        ],
        out_specs=pl.BlockSpec((_B_BLK, act_dim), lambda i: (i, 0)),
        compiler_params=pltpu.CompilerParams(
            dimension_semantics=("parallel",)),
        cost_estimate=pl.CostEstimate(flops=flops, transcendentals=0,
                                      bytes_accessed=bytes_accessed),
    )(x_t, w1cat, sh1rep, s1sel, w2cat, sh2rep, c20, c21,
      wp, bfc, wf1, bf1, wf2, bf2)
    return out
